# Initial kernel scaffold; baseline (speedup 1.0000x reference)
import jax
import jax.numpy as jnp
from jax.experimental import pallas as pl


def kernel(data, ids, space_pts, time_pts, query_pts, params):
    raise NotImplementedError("write your pallas kernel here")


# speedup vs baseline: 1.0005x; 1.0005x over previous
import jax
import jax.numpy as jnp
from jax.experimental import pallas as pl

K = 16


def _mlp(x, layers):
    n = len(layers)
    for i, (W, b) in enumerate(layers):
        x = x @ W + b
        if i < n - 1:
            x = jax.nn.relu(x)
    return x


def _knn_idx(q, s, k):
    d2 = jnp.sum(q * q, -1)[:, :, None] + jnp.sum(s * s, -1)[:, None, :] - 2.0 * jnp.einsum('bqd,bsd->bqs', q, s)
    _, idx = jax.lax.top_k(-d2, k)
    return idx


def _gather(x, idx):
    return jax.vmap(lambda xb, ib: xb[ib])(x, idx)


def _point_conv_from_idx(idx, q_pts, s_pts, feats, p):
    nb_pts = _gather(s_pts, idx)
    rel = nb_pts - q_pts[:, :, None, :]
    w = _mlp(rel, p["w"])
    nb_feats = _gather(feats, idx)
    M = jnp.einsum('bqkm,bqkc->bqmc', w, nb_feats)
    M = M.reshape(M.shape[0], M.shape[1], -1)
    return _mlp(M, p["f"])


def _copy_kernel(x_ref, o_ref):
    o_ref[...] = x_ref[...]


def kernel(data, ids, space_pts, time_pts, query_pts, params):
    sp_idx = _knn_idx(space_pts, space_pts, K)
    ti_idx = _knn_idx(time_pts, time_pts, K)
    q_idx = _knn_idx(query_pts, time_pts, K)

    x = data
    for i in range(len(params["space"])):
        sp = _point_conv_from_idx(sp_idx, space_pts, space_pts, x, params["space"][i])
        ti = _point_conv_from_idx(ti_idx, time_pts, time_pts,
                                  jnp.concatenate([x, sp], axis=2), params["time"][i])
        x = _mlp(jnp.concatenate([x, sp, ti], axis=2), params["comb"][i])
    out = _point_conv_from_idx(q_idx, query_pts, time_pts, x, params["target"])
    out = pl.pallas_call(
        _copy_kernel,
        out_shape=jax.ShapeDtypeStruct(out.shape, out.dtype),
    )(out)
    return out


# speedup vs baseline: 2.1725x; 2.1714x over previous
import functools

import jax
import jax.numpy as jnp
from jax.experimental import pallas as pl
from jax.experimental.pallas import tpu as pltpu

K = 16


def _mlp(x, layers):
    n = len(layers)
    for i, (W, b) in enumerate(layers):
        x = x @ W + b
        if i < n - 1:
            x = jax.nn.relu(x)
    return x


def _gather(x, idx):
    return jax.vmap(lambda xb, ib: xb[ib])(x, idx)


def _knn_kernel(q_ref, s_ref, o_ref, *, n_s, k):
    q = q_ref[0]
    s = s_ref[0]
    q2 = jnp.sum(q * q, axis=1)
    s2 = jnp.sum(s * s, axis=1)
    qs = jax.lax.dot_general(s, q, (((1,), (1,)), ((), ())),
                             preferred_element_type=jnp.float32)
    d2 = s2[:, None] + q2[None, :] - 2.0 * qs
    iota = jax.lax.broadcasted_iota(jnp.int32, d2.shape, 0)
    big = jnp.float32(jnp.inf)
    for kk in range(k):
        m = jnp.min(d2, axis=0, keepdims=True)
        ii = jnp.min(jnp.where(d2 == m, iota, n_s), axis=0,
                     keepdims=True)
        o_ref[0, kk, :] = ii[0, :]
        d2 = jnp.where(iota == ii, big, d2)


def _knn_idx_pallas(q, s, k, qb):
    b, nq, d = q.shape
    s_n = s.shape[1]
    grid = (b, nq // qb)
    out = pl.pallas_call(
        functools.partial(_knn_kernel, n_s=s_n, k=k),
        grid=grid,
        in_specs=[
            pl.BlockSpec((1, qb, d), lambda i, j: (i, j, 0)),
            pl.BlockSpec((1, s_n, d), lambda i, j: (i, 0, 0)),
        ],
        out_specs=pl.BlockSpec((1, k, qb), lambda i, j: (i, 0, j)),
        out_shape=jax.ShapeDtypeStruct((b, k, nq), jnp.int32),
        compiler_params=pltpu.CompilerParams(
            dimension_semantics=("parallel", "parallel"),
        ),
    )(q, s)
    return jnp.transpose(out, (0, 2, 1))


def _point_conv_from_idx(idx, q_pts, s_pts, feats, p):
    nb_pts = _gather(s_pts, idx)
    rel = nb_pts - q_pts[:, :, None, :]
    w = _mlp(rel, p["w"])
    nb_feats = _gather(feats, idx)
    M = jnp.einsum('bqkm,bqkc->bqmc', w, nb_feats)
    M = M.reshape(M.shape[0], M.shape[1], -1)
    return _mlp(M, p["f"])


def kernel(data, ids, space_pts, time_pts, query_pts, params):
    sp_idx = _knn_idx_pallas(space_pts, space_pts, K, 256)
    ti_idx = _knn_idx_pallas(time_pts, time_pts, K, 256)
    q_idx = _knn_idx_pallas(query_pts, time_pts, K, 256)

    x = data
    for i in range(len(params["space"])):
        sp = _point_conv_from_idx(sp_idx, space_pts, space_pts, x, params["space"][i])
        ti = _point_conv_from_idx(ti_idx, time_pts, time_pts,
                                  jnp.concatenate([x, sp], axis=2), params["time"][i])
        x = _mlp(jnp.concatenate([x, sp, ti], axis=2), params["comb"][i])
    return _point_conv_from_idx(q_idx, query_pts, time_pts, x, params["target"])


# speedup vs baseline: 9.7637x; 4.4943x over previous
import functools

import jax
import jax.numpy as jnp
from jax.experimental import pallas as pl
from jax.experimental.pallas import tpu as pltpu
from jax.experimental.pallas import tpu_sc as plsc

K = 16


def _mlp(x, layers):
    n = len(layers)
    for i, (W, b) in enumerate(layers):
        x = x @ W + b
        if i < n - 1:
            x = jax.nn.relu(x)
    return x


def _gather(x, idx):
    return jax.vmap(lambda xb, ib: xb[ib])(x, idx)


def _knn_kernel(q_ref, s_ref, o_ref, *, n_s, k):
    q = q_ref[0]
    s = s_ref[0]
    q2 = jnp.sum(q * q, axis=1)
    s2 = jnp.sum(s * s, axis=1)
    qs = jax.lax.dot_general(s, q, (((1,), (1,)), ((), ())),
                             preferred_element_type=jnp.float32)
    d2 = s2[:, None] + q2[None, :] - 2.0 * qs
    iota = jax.lax.broadcasted_iota(jnp.int32, d2.shape, 0)
    big = jnp.float32(jnp.inf)
    for kk in range(k):
        m = jnp.min(d2, axis=0, keepdims=True)
        ii = jnp.min(jnp.where(d2 == m, iota, n_s), axis=0,
                     keepdims=True)
        o_ref[0, kk, :] = ii[0, :]
        d2 = jnp.where(iota == ii, big, d2)


def _knn_idx_pallas(q, s, k, qb):
    b, nq, d = q.shape
    s_n = s.shape[1]
    grid = (b, nq // qb)
    out = pl.pallas_call(
        functools.partial(_knn_kernel, n_s=s_n, k=k),
        grid=grid,
        in_specs=[
            pl.BlockSpec((1, qb, d), lambda i, j: (i, j, 0)),
            pl.BlockSpec((1, s_n, d), lambda i, j: (i, 0, 0)),
        ],
        out_specs=pl.BlockSpec((1, k, qb), lambda i, j: (i, 0, j)),
        out_shape=jax.ShapeDtypeStruct((b, k, nq), jnp.int32),
        compiler_params=pltpu.CompilerParams(
            dimension_semantics=("parallel", "parallel"),
        ),
    )(q, s)
    return jnp.transpose(out, (0, 2, 1))


_GATHER_WINDOW = 128


def _sc_gather(table, flat_idx):
    n_idx = flat_idx.shape[0]
    c = table.shape[1]
    mesh = plsc.VectorSubcoreMesh(core_axis_name="core", subcore_axis_name="subcore")
    idx2 = flat_idx.reshape(1, n_idx)

    @pl.kernel(out_type=jax.ShapeDtypeStruct((n_idx, c), table.dtype), mesh=mesh)
    def gk(x_hbm, i_hbm, o_hbm):
        def body(i_vmem, o_vmem):
            pltpu.sync_copy(x_hbm.at[i_vmem.at[0]], o_vmem)

        pltpu.emit_pipeline(
            body,
            grid=(n_idx // _GATHER_WINDOW,),
            in_specs=[pl.BlockSpec((1, _GATHER_WINDOW), index_map=lambda i: (0, i))],
            out_specs=[pl.BlockSpec((_GATHER_WINDOW, c), index_map=lambda i: (i, 0))],
            core_axis_name="subcore",
            dimension_semantics=(pltpu.PARALLEL,),
        )(i_hbm, o_hbm)

    return gk(table, idx2)


def _point_conv_sc(flat_idx, q_pts, s_pts, feats, p):
    b, nq = q_pts.shape[0], q_pts.shape[1]
    c_in = feats.shape[2]
    dim = s_pts.shape[2]
    table = jnp.concatenate([feats, s_pts], axis=2).reshape(b * feats.shape[1], c_in + dim)
    table = jnp.pad(table, ((0, 0), (0, 128 - table.shape[1])))
    g = _sc_gather(table, flat_idx).reshape(b, nq, K, 128)
    nb_feats = g[..., :c_in]
    nb_pts = g[..., c_in:c_in + dim]
    rel = nb_pts - q_pts[:, :, None, :]
    w = _mlp(rel, p["w"])
    M = jnp.einsum('bqkm,bqkc->bqmc', w, nb_feats)
    M = M.reshape(M.shape[0], M.shape[1], -1)
    return _mlp(M, p["f"])


def _flat_idx(idx, n_rows):
    b = idx.shape[0]
    off = (jnp.arange(b, dtype=jnp.int32) * n_rows)[:, None, None]
    return (idx + off).reshape(-1)


def kernel(data, ids, space_pts, time_pts, query_pts, params):
    n = space_pts.shape[1]
    sp_idx = _flat_idx(_knn_idx_pallas(space_pts, space_pts, K, 256), n)
    ti_idx = _flat_idx(_knn_idx_pallas(time_pts, time_pts, K, 256), n)
    q_idx = _flat_idx(_knn_idx_pallas(query_pts, time_pts, K, 256), n)

    x = data
    for i in range(len(params["space"])):
        sp = _point_conv_sc(sp_idx, space_pts, space_pts, x, params["space"][i])
        ti = _point_conv_sc(ti_idx, time_pts, time_pts,
                            jnp.concatenate([x, sp], axis=2), params["time"][i])
        x = _mlp(jnp.concatenate([x, sp, ti], axis=2), params["comb"][i])
    return _point_conv_sc(q_idx, query_pts, time_pts, x, params["target"])
